# Initial kernel scaffold; baseline (speedup 1.0000x reference)
import jax
import jax.numpy as jnp
from jax.experimental import pallas as pl


def kernel(idxs):
    raise NotImplementedError("write your pallas kernel here")


# speedup vs baseline: 1.7279x; 1.7279x over previous
import jax
import jax.numpy as jnp
from jax.experimental import pallas as pl
from jax.experimental.pallas import tpu as pltpu

_B = 16384
_C = 1000
_BR = 512
_NB = _B // _BR


def _onehot_block(idx_ref, out_ref):
    cols = jax.lax.broadcasted_iota(jnp.int32, out_ref.shape, 1)
    out_ref[...] = jnp.where(idx_ref[...] == cols, 1.0, 0.0)


def kernel(idxs):
    idxs2 = idxs.reshape(_B, 1).astype(jnp.int32)
    return pl.pallas_call(
        _onehot_block,
        grid=(_NB,),
        in_specs=[pl.BlockSpec((_BR, 1), lambda i: (i, 0))],
        out_specs=pl.BlockSpec((_BR, _C), lambda i: (i, 0)),
        out_shape=jax.ShapeDtypeStruct((_B, _C), jnp.float32),
        compiler_params=pltpu.CompilerParams(
            dimension_semantics=("parallel",),
        ),
    )(idxs2)


# speedup vs baseline: 1.9695x; 1.1398x over previous
import jax
import jax.numpy as jnp
from jax.experimental import pallas as pl
from jax.experimental.pallas import tpu as pltpu

_B = 16384
_C = 1000
_BR = 512
_NB = _B // _BR


def _onehot_block(idx_ref, out_ref):
    idx = idx_ref[0, 0, :].reshape(_BR, 1)
    cols = jax.lax.broadcasted_iota(jnp.int32, out_ref.shape, 1)
    out_ref[...] = jnp.where(idx == cols, 1.0, 0.0)


def kernel(idxs):
    idxs3 = idxs.astype(jnp.int32).reshape(_NB, 1, _BR)
    return pl.pallas_call(
        _onehot_block,
        grid=(_NB,),
        in_specs=[pl.BlockSpec((1, 1, _BR), lambda i: (i, 0, 0))],
        out_specs=pl.BlockSpec((_BR, _C), lambda i: (i, 0)),
        out_shape=jax.ShapeDtypeStruct((_B, _C), jnp.float32),
        compiler_params=pltpu.CompilerParams(
            dimension_semantics=("parallel",),
        ),
    )(idxs3)


# speedup vs baseline: 2.0970x; 1.0648x over previous
import jax
import jax.numpy as jnp
from jax.experimental import pallas as pl
from jax.experimental.pallas import tpu as pltpu

_B = 16384
_C = 1000
_BR = 2048
_NB = _B // _BR


def _onehot_block(idx_ref, out_ref):
    idx = idx_ref[0, 0, :].reshape(_BR, 1)
    cols = jax.lax.broadcasted_iota(jnp.int32, out_ref.shape, 1)
    out_ref[...] = jnp.where(idx == cols, 1.0, 0.0)


def kernel(idxs):
    idxs3 = idxs.astype(jnp.int32).reshape(_NB, 1, _BR)
    return pl.pallas_call(
        _onehot_block,
        grid=(_NB,),
        in_specs=[pl.BlockSpec((1, 1, _BR), lambda i: (i, 0, 0))],
        out_specs=pl.BlockSpec((_BR, _C), lambda i: (i, 0)),
        out_shape=jax.ShapeDtypeStruct((_B, _C), jnp.float32),
        compiler_params=pltpu.CompilerParams(
            dimension_semantics=("parallel",),
        ),
    )(idxs3)
